# Initial kernel scaffold; baseline (speedup 1.0000x reference)
import jax
import jax.numpy as jnp
from jax.experimental import pallas as pl


def kernel(x, importance, W_proj, b_proj, neuron_emb):
    raise NotImplementedError("write your pallas kernel here")


# speedup vs baseline: 32.9229x; 32.9229x over previous
import jax
import jax.numpy as jnp
from jax.experimental import pallas as pl
from jax.experimental.pallas import tpu as pltpu

_B, _S, _D_MODEL, _D_SPACE = 4, 2048, 4096, 64
_N_POOL = 512
_RV_END = _N_POOL * 6
_TOPKS = (8, 8, 3, 8, 8, 3)
_TILE = 256


def _router_kernel(x_ref, w_ref, b_ref, emb_ref, out_ref):
    x = x_ref[...]
    w = w_ref[...]
    proj = jax.lax.dot_general(
        x, w, (((1,), (0,)), ((), ())), preferred_element_type=jnp.float32
    )
    proj = proj + b_ref[...]
    emb = emb_ref[...]
    inv_norm = 1.0 / jnp.maximum(
        jnp.sqrt(jnp.sum(emb * emb, axis=1, keepdims=True)), 1e-12
    )
    emb_n = emb * inv_norm
    neg = jnp.float32(-jnp.inf)
    for g in range(6):
        h = proj[:, g * _D_SPACE:(g + 1) * _D_SPACE]
        e = emb_n[g * _N_POOL:(g + 1) * _N_POOL, :]
        logits = jax.lax.dot_general(
            h, e, (((1,), (1,)), ((), ())), preferred_element_type=jnp.float32
        )
        m = jnp.max(logits, axis=1, keepdims=True)
        ex = jnp.exp(logits - m)
        z = jnp.sum(ex, axis=1, keepdims=True)
        p = ex / z
        vals = logits
        t = m
        for _ in range(_TOPKS[g]):
            t = jnp.max(vals, axis=1, keepdims=True)
            vals = jnp.where(vals >= t, neg, vals)
        out_ref[:, g * _N_POOL:(g + 1) * _N_POOL] = jnp.where(logits >= t, p, 0.0)


def kernel(x, importance, W_proj, b_proj, neuron_emb):
    del importance
    xf = x.reshape(_B * _S, _D_MODEL)
    emb = neuron_emb[:_RV_END]
    b2 = b_proj.reshape(1, _D_SPACE * 6)
    out = pl.pallas_call(
        _router_kernel,
        grid=(_B * _S // _TILE,),
        in_specs=[
            pl.BlockSpec((_TILE, _D_MODEL), lambda i: (i, 0)),
            pl.BlockSpec((_D_MODEL, _D_SPACE * 6), lambda i: (0, 0)),
            pl.BlockSpec((1, _D_SPACE * 6), lambda i: (0, 0)),
            pl.BlockSpec((_RV_END, _D_SPACE), lambda i: (0, 0)),
        ],
        out_specs=pl.BlockSpec((_TILE, _RV_END), lambda i: (i, 0)),
        out_shape=jax.ShapeDtypeStruct((_B * _S, _RV_END), jnp.float32),
        compiler_params=pltpu.CompilerParams(dimension_semantics=("arbitrary",)),
    )(xf, W_proj, b2, emb)
    return out.reshape(_B, _S, _RV_END)
